# baseline (speedup 1.0000x reference)
import functools

import jax
import jax.numpy as jnp
from jax import lax
from jax.experimental import pallas as pl
from jax.experimental.pallas import tpu as pltpu
from jax.experimental.pallas import tpu_sc as plsc

BATCH = 4096
MAXLEN = 200
EMBED = 64
HALF = MAXLEN // 2
NUM_CORES = 2
NUM_SUBCORES = 16
NW = NUM_CORES * NUM_SUBCORES
BPW = BATCH // NW
LANES = 16
NBUF = 4
LAG = 2


def _emb_body(x_hbm, tok_hbm, pos_hbm, out_hbm, idx_v, pos_v, bufs, gsems, osems):
    wid = lax.axis_index("s") * NUM_CORES + lax.axis_index("c")
    base = wid * BPW

    pltpu.sync_copy(x_hbm.at[pl.ds(base, BPW)], idx_v)
    pltpu.sync_copy(pos_hbm, pos_v)

    def fire_gather(t, k):
        for j in range(2):
            pltpu.async_copy(
                tok_hbm.at[idx_v.at[t, j]],
                bufs.at[k, pl.ds(j * HALF, HALF)],
                gsems.at[k],
            )

    def wait_gather(t, k):
        for j in range(2):
            pltpu.make_async_copy(
                tok_hbm.at[idx_v.at[t, j]],
                bufs.at[k, pl.ds(j * HALF, HALF)],
                gsems.at[k],
            ).wait()

    def add_pos(k):
        def addrow(r, carry):
            for c in range(EMBED // LANES):
                sl = pl.ds(c * LANES, LANES)
                plsc.addupdate(bufs.at[k, r, sl], pos_v[r, sl])
            return carry

        lax.fori_loop(0, MAXLEN, addrow, 0, unroll=8)

    def fire_out(t, k):
        pltpu.async_copy(
            bufs.at[k], out_hbm.at[pl.ds((base + t) * MAXLEN, MAXLEN)], osems.at[k]
        )

    def wait_out(k):
        pltpu.make_async_copy(
            bufs.at[k], out_hbm.at[pl.ds(base * MAXLEN, MAXLEN)], osems.at[k]
        ).wait()

    for t in range(LAG):
        fire_gather(t, t % NBUF)

    def block(blk, carry):
        g = blk * NBUF
        for b in range(NBUF):
            t = g + b
            kf = (b + LAG) % NBUF
            tf = t + LAG

            @pl.when(tf < BPW)
            def _():
                @pl.when(tf >= NBUF)
                def _():
                    wait_out(kf)

                fire_gather(tf, kf)

            wait_gather(t, b)
            add_pos(b)
            fire_out(t, b)
        return carry

    lax.fori_loop(0, BPW // NBUF, block, 0)

    for k in range(NBUF):
        wait_out(k)


_emb = functools.partial(
    pl.kernel,
    mesh=plsc.VectorSubcoreMesh(core_axis_name="c", subcore_axis_name="s"),
    out_type=jax.ShapeDtypeStruct((BATCH * MAXLEN, EMBED), jnp.float32),
    scratch_types=[
        pltpu.VMEM((BPW, 2, HALF), jnp.int32),
        pltpu.VMEM((MAXLEN, EMBED), jnp.float32),
        pltpu.VMEM((NBUF, MAXLEN, EMBED), jnp.float32),
        pltpu.SemaphoreType.DMA((NBUF,)),
        pltpu.SemaphoreType.DMA((NBUF,)),
    ],
    compiler_params=pltpu.CompilerParams(use_tc_tiling_on_sc=False),
)(_emb_body)


VOCAB = 1000000
HALFV = 524288
TBLK = 512
TGRID = HALFV // TBLK
_LAST_B = (VOCAB - 1) // TBLK


def _pack_body(a_ref, b_ref, o_ref):
    o_ref[...] = jnp.concatenate([a_ref[...].T, b_ref[...].T], axis=1)


def _pack_table(tt):
    return pl.pallas_call(
        _pack_body,
        grid=(TGRID,),
        in_specs=[
            pl.BlockSpec((EMBED, TBLK), lambda i: (0, i)),
            pl.BlockSpec((EMBED, TBLK), lambda i: (0, jnp.minimum(i + TGRID, _LAST_B))),
        ],
        out_specs=pl.BlockSpec((TBLK, 2 * EMBED), lambda i: (i, 0)),
        out_shape=jax.ShapeDtypeStruct((HALFV, 2 * EMBED), jnp.float32),
    )(tt, tt)


def kernel(x, token_table, pos_table):
    packed = _pack_table(token_table.T).reshape(2 * HALFV, EMBED)
    xr = jnp.where(x < HALFV, 2 * x, 2 * (x - HALFV) + 1).astype(jnp.int32)
    x3 = xr.reshape(BATCH, 2, HALF)
    out = _emb(x3, packed, pos_table)
    return out.reshape(BATCH, MAXLEN, EMBED)


# speedup vs baseline: 1.1181x; 1.1181x over previous
import functools

import jax
import jax.numpy as jnp
from jax import lax
from jax.experimental import pallas as pl
from jax.experimental.pallas import tpu as pltpu
from jax.experimental.pallas import tpu_sc as plsc

BATCH = 4096
MAXLEN = 200
EMBED = 64
VOCAB = 1000000
HALF = MAXLEN // 2
NUM_CORES = 2
NUM_SUBCORES = 16
NW = NUM_CORES * NUM_SUBCORES
BPW = BATCH // NW
LANES = 16
NBUF = 4
LAG = 2


def _emb_body(x_hbm, tok_hbm, pos_hbm, out_hbm, idx_v, pos_v, bufs, gsems, osems):
    wid = lax.axis_index("s") * NUM_CORES + lax.axis_index("c")
    base = wid * BPW

    pltpu.sync_copy(x_hbm.at[pl.ds(base, BPW)], idx_v)
    pltpu.sync_copy(pos_hbm, pos_v)

    def fire_gather(t, k):
        for j in range(2):
            pltpu.async_copy(
                tok_hbm.at[idx_v.at[t, j]],
                bufs.at[k, pl.ds(j * HALF, HALF)],
                gsems.at[k],
            )

    def wait_gather(t, k):
        for j in range(2):
            pltpu.make_async_copy(
                tok_hbm.at[idx_v.at[t, j]],
                bufs.at[k, pl.ds(j * HALF, HALF)],
                gsems.at[k],
            ).wait()

    def add_pos(k):
        def addrow(r, carry):
            for c in range(EMBED // LANES):
                sl = pl.ds(c * LANES, LANES)
                plsc.addupdate(bufs.at[k, r, sl], pos_v[r, sl])
            return carry

        lax.fori_loop(0, MAXLEN, addrow, 0, unroll=8)

    def fire_out(t, k):
        pltpu.async_copy(bufs.at[k], out_hbm.at[base + t], osems.at[k])

    def wait_out(k):
        pltpu.make_async_copy(bufs.at[k], out_hbm.at[base], osems.at[k]).wait()

    for t in range(LAG):
        fire_gather(t, t % NBUF)

    def block(blk, carry):
        g = blk * NBUF
        for b in range(NBUF):
            t = g + b
            kf = (b + LAG) % NBUF
            tf = t + LAG

            @pl.when(tf < BPW)
            def _():
                @pl.when(tf >= NBUF)
                def _():
                    wait_out(kf)

                fire_gather(tf, kf)

            wait_gather(t, b)
            add_pos(b)
            fire_out(t, b)
        return carry

    lax.fori_loop(0, BPW // NBUF, block, 0)

    for k in range(NBUF):
        wait_out(k)


_emb = functools.partial(
    pl.kernel,
    mesh=plsc.VectorSubcoreMesh(core_axis_name="c", subcore_axis_name="s"),
    out_type=jax.ShapeDtypeStruct((BATCH, MAXLEN, EMBED), jnp.float32),
    scratch_types=[
        pltpu.VMEM((BPW, 2, HALF), jnp.int32),
        pltpu.VMEM((MAXLEN, EMBED), jnp.float32),
        pltpu.VMEM((NBUF, MAXLEN, EMBED), jnp.float32),
        pltpu.SemaphoreType.DMA((NBUF,)),
        pltpu.SemaphoreType.DMA((NBUF,)),
    ],
    compiler_params=pltpu.CompilerParams(use_tc_tiling_on_sc=False),
)(_emb_body)


def kernel(x, token_table, pos_table):
    x3 = x.astype(jnp.int32).reshape(BATCH, 2, HALF)
    return _emb(x3, token_table, pos_table)


# speedup vs baseline: 1.3736x; 1.2284x over previous
import functools

import jax
import jax.numpy as jnp
from jax import lax
from jax.experimental import pallas as pl
from jax.experimental.pallas import tpu as pltpu
from jax.experimental.pallas import tpu_sc as plsc

BATCH = 4096
MAXLEN = 200
EMBED = 64
VOCAB = 1000000
HALF = MAXLEN // 2
NUM_CORES = 2
NUM_SUBCORES = 16
NW = NUM_CORES * NUM_SUBCORES
BPW = BATCH // NW
LANES = 16
NBUF = 4
LAG = 2


def _emb_body(x_hbm, tok_hbm, pos_hbm, out_hbm, idx_v, pos_v, bufs, gsems, osems):
    wid = lax.axis_index("s") * NUM_CORES + lax.axis_index("c")
    base = wid * BPW

    pltpu.sync_copy(x_hbm.at[pl.ds(base, BPW)], idx_v)
    pltpu.sync_copy(pos_hbm, pos_v)

    def fire_gather(t, k):
        for j in range(2):
            pltpu.async_copy(
                tok_hbm.at[idx_v.at[t, j]],
                bufs.at[k, pl.ds(j * HALF, HALF)],
                gsems.at[k],
            )

    def wait_gather(t, k):
        for j in range(2):
            pltpu.make_async_copy(
                tok_hbm.at[idx_v.at[t, j]],
                bufs.at[k, pl.ds(j * HALF, HALF)],
                gsems.at[k],
            ).wait()

    def add_pos(k):
        def addrow(r, carry):
            for c in range(EMBED // LANES):
                sl = pl.ds(c * LANES, LANES)
                plsc.addupdate(bufs.at[k, r, sl], pos_v[r, sl])
            return carry

        lax.fori_loop(0, MAXLEN, addrow, 0, unroll=8)

    def fire_out(t, k):
        pltpu.async_copy(
            bufs.at[k],
            out_hbm.at[pl.ds((base + t) * MAXLEN, MAXLEN), pl.ds(0, EMBED)],
            osems.at[k],
        )

    def wait_out(k):
        pltpu.make_async_copy(
            bufs.at[k],
            out_hbm.at[pl.ds(base * MAXLEN, MAXLEN), pl.ds(0, EMBED)],
            osems.at[k],
        ).wait()

    for t in range(LAG):
        fire_gather(t, t % NBUF)

    def block(blk, carry):
        g = blk * NBUF
        for b in range(NBUF):
            t = g + b
            kf = (b + LAG) % NBUF
            tf = t + LAG

            @pl.when(tf < BPW)
            def _():
                @pl.when(tf >= NBUF)
                def _():
                    wait_out(kf)

                fire_gather(tf, kf)

            wait_gather(t, b)
            add_pos(b)
            fire_out(t, b)
        return carry

    lax.fori_loop(0, BPW // NBUF, block, 0)

    for k in range(NBUF):
        wait_out(k)


_emb = functools.partial(
    pl.kernel,
    mesh=plsc.VectorSubcoreMesh(core_axis_name="c", subcore_axis_name="s"),
    out_type=jax.ShapeDtypeStruct((BATCH * MAXLEN, 2 * EMBED), jnp.float32),
    scratch_types=[
        pltpu.VMEM((BPW, 2, HALF), jnp.int32),
        pltpu.VMEM((MAXLEN, EMBED), jnp.float32),
        pltpu.VMEM((NBUF, MAXLEN, EMBED), jnp.float32),
        pltpu.SemaphoreType.DMA((NBUF,)),
        pltpu.SemaphoreType.DMA((NBUF,)),
    ],
    compiler_params=pltpu.CompilerParams(use_tc_tiling_on_sc=False),
)(_emb_body)


HALFV = VOCAB // 2
TBLK = 2000
TGRID = HALFV // TBLK


def _pack_body(a_ref, b_ref, o_ref):
    o_ref[...] = jnp.concatenate([a_ref[...], b_ref[...]], axis=1)


def _pack_table(tt):
    return pl.pallas_call(
        _pack_body,
        grid=(TGRID,),
        in_specs=[
            pl.BlockSpec((TBLK, EMBED), lambda i: (i, 0)),
            pl.BlockSpec((TBLK, EMBED), lambda i: (i + TGRID, 0)),
        ],
        out_specs=pl.BlockSpec((TBLK, 2 * EMBED), lambda i: (i, 0)),
        out_shape=jax.ShapeDtypeStruct((HALFV, 2 * EMBED), jnp.float32),
    )(tt, tt)


def kernel(x, token_table, pos_table):
    packed = _pack_table(token_table).reshape(VOCAB, EMBED)
    xr = jnp.where(x < HALFV, 2 * x, 2 * (x - HALFV) + 1).astype(jnp.int32)
    x3 = xr.reshape(BATCH, 2, HALF)
    out = _emb(x3, packed, pos_table)
    return out.reshape(BATCH, MAXLEN, 2 * EMBED)[:, :, :EMBED]
